# Initial kernel scaffold; baseline (speedup 1.0000x reference)
import jax
import jax.numpy as jnp
from jax.experimental import pallas as pl


def kernel(x, batch, W, b):
    raise NotImplementedError("write your pallas kernel here")


# speedup vs baseline: 6.4436x; 6.4436x over previous
import jax
import jax.numpy as jnp
from jax.experimental import pallas as pl
from jax.experimental.pallas import tpu as pltpu

N_NODES = 100000
IN_FEATS = 512
NUM_SEGMENTS = 1024
BLOCK_ROWS = 2000
NUM_BLOCKS = N_NODES // BLOCK_ROWS


def _wsum_kernel(batch_ref, x_ref, w_ref, b_ref, out_ref):
    i = pl.program_id(0)

    @pl.when(i == 0)
    def _init():
        out_ref[...] = jnp.zeros_like(out_ref)

    xb = x_ref[...]
    z = jnp.dot(xb, w_ref[...], preferred_element_type=jnp.float32)
    w = jax.nn.sigmoid(z + b_ref[0, 0])
    xw = xb * w

    ids = batch_ref[0, 0, :]
    seg_iota = jax.lax.broadcasted_iota(jnp.int32, (NUM_SEGMENTS, BLOCK_ROWS), 0)
    onehot = jnp.where(seg_iota == ids[None, :], 1.0, 0.0).astype(jnp.float32)
    out_ref[...] += jnp.dot(onehot, xw, preferred_element_type=jnp.float32)


def kernel(x, batch, W, b):
    batch3 = batch.reshape(NUM_BLOCKS, 1, BLOCK_ROWS)
    b2 = b.reshape(1, 1)
    out = pl.pallas_call(
        _wsum_kernel,
        grid=(NUM_BLOCKS,),
        in_specs=[
            pl.BlockSpec((1, 1, BLOCK_ROWS), lambda i: (i, 0, 0)),
            pl.BlockSpec((BLOCK_ROWS, IN_FEATS), lambda i: (i, 0)),
            pl.BlockSpec((IN_FEATS, 1), lambda i: (0, 0)),
            pl.BlockSpec(memory_space=pltpu.SMEM),
        ],
        out_specs=pl.BlockSpec((NUM_SEGMENTS, IN_FEATS), lambda i: (0, 0)),
        out_shape=jax.ShapeDtypeStruct((NUM_SEGMENTS, IN_FEATS), jnp.float32),
    )(batch3, x, W, b2)
    return out
